# baseline (speedup 1.0000x reference)
import functools

import jax
import jax.numpy as jnp
from jax import lax
from jax.experimental import pallas as pl
from jax.experimental.pallas import tpu as pltpu
from jax.experimental.pallas import tpu_sc as plsc

V = 100000
D = 16
B = 1024

VT = 2048
NV = (V + VT - 1) // VT
NC = 2
NJ = 25
VP = NC * NJ * VT


def _sc_gather(table_flat, inputs):
    info = plsc.get_sparse_core_info()
    nw = info.num_cores * info.num_subcores
    bpw = B // nw
    npw = bpw * D

    mesh = plsc.VectorSubcoreMesh(core_axis_name="c", subcore_axis_name="s")

    @functools.partial(
        pl.kernel,
        mesh=mesh,
        out_type=jax.ShapeDtypeStruct((B * D,), jnp.float32),
        scratch_types=[
            pltpu.VMEM((bpw,), jnp.int32),
            pltpu.VMEM((npw,), jnp.int32),
            pltpu.VMEM((npw,), jnp.float32),
            pltpu.SemaphoreType.DMA,
        ],
        compiler_params=pltpu.CompilerParams(use_tc_tiling_on_sc=False),
    )
    def gather_kernel(flat_hbm, idx_hbm, out_hbm, idx_v, fidx_v, rows_v, sem):
        wid = lax.axis_index("s") * info.num_cores + lax.axis_index("c")
        base = wid * bpw
        pltpu.sync_copy(idx_hbm.at[pl.ds(base, bpw)], idx_v)
        for dd in range(D):
            for h in range(bpw // 16):
                fidx_v[pl.ds(dd * bpw + h * 16, 16)] = (
                    idx_v[pl.ds(h * 16, 16)] + dd * V)
        copies = [
            pltpu.async_copy(
                flat_hbm.at[fidx_v.at[pl.ds(i * 128, 128)]],
                rows_v.at[pl.ds(i * 128, 128)], sem)
            for i in range(npw // 128)
        ]
        for c in copies:
            c.wait()
        for dd in range(D):
            pltpu.sync_copy(rows_v.at[pl.ds(dd * bpw, bpw)],
                            out_hbm.at[pl.ds(dd * B + base, bpw)])

    return gather_kernel(table_flat, inputs).reshape(D, B)


def _scores_tile(emb_ref, w_ref, b_ref):
    et = emb_ref[...].astype(jnp.bfloat16)
    wt = w_ref[...].astype(jnp.bfloat16)
    scores = lax.dot_general(
        wt, et, (((0,), (0,)), ((), ())),
        preferred_element_type=jnp.float32)
    ones = jnp.ones((1, B), jnp.bfloat16)
    bias = lax.dot_general(
        b_ref[...].astype(jnp.bfloat16), ones, (((0,), (0,)), ((), ())),
        preferred_element_type=jnp.float32)
    return scores + bias


def _pass1_body(emb_ref, w_ref, b_ref, m_ref, s_ref, m_scr, s_scr):
    j = pl.program_id(1)

    @pl.when(j == 0)
    def _():
        m_scr[...] = jnp.full((1, B), -jnp.inf, jnp.float32)
        s_scr[...] = jnp.zeros((1, B), jnp.float32)

    scores = _scores_tile(emb_ref, w_ref, b_ref)

    m_old = m_scr[...]
    m_new = jnp.maximum(m_old, jnp.max(scores, axis=0, keepdims=True))
    s = (s_scr[...] * jnp.exp(m_old - m_new)
         + jnp.sum(jnp.exp(scores - m_new), axis=0, keepdims=True))
    m_scr[...] = m_new
    s_scr[...] = s

    @pl.when(j == NJ - 1)
    def _():
        m_ref[...] = m_new[None]
        s_ref[...] = s[None]


def _pass1(embeds, w, b2):
    m, s = pl.pallas_call(
        _pass1_body,
        grid=(NC, NJ),
        in_specs=[
            pl.BlockSpec((D, B), lambda i, j: (0, 0)),
            pl.BlockSpec((D, VT), lambda i, j: (0, i * NJ + j)),
            pl.BlockSpec((1, VT), lambda i, j: (0, i * NJ + j)),
        ],
        out_specs=[
            pl.BlockSpec((1, 1, B), lambda i, j: (i, 0, 0)),
            pl.BlockSpec((1, 1, B), lambda i, j: (i, 0, 0)),
        ],
        out_shape=[
            jax.ShapeDtypeStruct((NC, 1, B), jnp.float32),
            jax.ShapeDtypeStruct((NC, 1, B), jnp.float32),
        ],
        scratch_shapes=[
            pltpu.VMEM((1, B), jnp.float32),
            pltpu.VMEM((1, B), jnp.float32),
        ],
        compiler_params=pltpu.CompilerParams(
            dimension_semantics=("parallel", "arbitrary")),
    )(embeds, w, b2)
    m = m[:, 0, :]
    s = s[:, 0, :]
    mg = jnp.max(m, axis=0, keepdims=True)
    return mg + jnp.log(jnp.sum(s * jnp.exp(m - mg), axis=0, keepdims=True))


def _pass2_body(emb_ref, w_ref, b_ref, norm_ref, out_ref):
    scores = _scores_tile(emb_ref, w_ref, b_ref)
    out_ref[...] = scores - norm_ref[...]


def _pass2(embeds, w, b2, norm):
    return pl.pallas_call(
        _pass2_body,
        grid=(NV,),
        in_specs=[
            pl.BlockSpec((D, B), lambda j: (0, 0)),
            pl.BlockSpec((D, VT), lambda j: (0, j)),
            pl.BlockSpec((1, VT), lambda j: (0, j)),
            pl.BlockSpec((1, B), lambda j: (0, 0)),
        ],
        out_specs=pl.BlockSpec((VT, B), lambda j: (j, 0)),
        out_shape=jax.ShapeDtypeStruct((V, B), jnp.float32),
    )(embeds, w, b2, norm)


def kernel(inputs, emb_table, W, b):
    embeds = _sc_gather(emb_table.T.reshape(-1), inputs)
    wt = jnp.pad(W.T, ((0, 0), (0, VP - V)))
    b2 = jnp.pad(b.reshape(1, V), ((0, 0), (0, VP - V)),
                 constant_values=-1e30)
    norm = _pass1(embeds, wt, b2)
    out_t = _pass2(embeds, wt, b2, norm)
    return out_t.T


# speedup vs baseline: 1.0214x; 1.0214x over previous
import functools

import jax
import jax.numpy as jnp
from jax import lax
from jax.experimental import pallas as pl
from jax.experimental.pallas import tpu as pltpu
from jax.experimental.pallas import tpu_sc as plsc

V = 100000
D = 16
B = 1024

VT = 4096
NV = (V + VT - 1) // VT
NP = NV
VP = NP * VT


def _sc_gather(table_flat, inputs):
    info = plsc.get_sparse_core_info()
    nw = info.num_cores * info.num_subcores
    bpw = B // nw
    npw = bpw * D

    mesh = plsc.VectorSubcoreMesh(core_axis_name="c", subcore_axis_name="s")

    @functools.partial(
        pl.kernel,
        mesh=mesh,
        out_type=jax.ShapeDtypeStruct((B * D,), jnp.float32),
        scratch_types=[
            pltpu.VMEM((bpw,), jnp.int32),
            pltpu.VMEM((npw,), jnp.int32),
            pltpu.VMEM((npw,), jnp.float32),
            pltpu.SemaphoreType.DMA,
        ],
        compiler_params=pltpu.CompilerParams(use_tc_tiling_on_sc=False),
    )
    def gather_kernel(flat_hbm, idx_hbm, out_hbm, idx_v, fidx_v, rows_v, sem):
        wid = lax.axis_index("s") * info.num_cores + lax.axis_index("c")
        base = wid * bpw
        pltpu.sync_copy(idx_hbm.at[pl.ds(base, bpw)], idx_v)
        for dd in range(D):
            for h in range(bpw // 16):
                fidx_v[pl.ds(dd * bpw + h * 16, 16)] = (
                    idx_v[pl.ds(h * 16, 16)] + dd * V)
        copies = [
            pltpu.async_copy(
                flat_hbm.at[fidx_v.at[pl.ds(i * 128, 128)]],
                rows_v.at[pl.ds(i * 128, 128)], sem)
            for i in range(npw // 128)
        ]
        for c in copies:
            c.wait()
        for dd in range(D):
            pltpu.sync_copy(rows_v.at[pl.ds(dd * bpw, bpw)],
                            out_hbm.at[pl.ds(dd * B + base, bpw)])

    return gather_kernel(table_flat, inputs).reshape(D, B)


def _scores_tile(emb_ref, w_ref, b_ref):
    et = emb_ref[...].astype(jnp.bfloat16)
    wt = w_ref[...].astype(jnp.bfloat16)
    scores = lax.dot_general(
        wt, et, (((0,), (0,)), ((), ())),
        preferred_element_type=jnp.float32)
    ones = jnp.ones((1, B), jnp.bfloat16)
    bias = lax.dot_general(
        b_ref[...].astype(jnp.bfloat16), ones, (((0,), (0,)), ((), ())),
        preferred_element_type=jnp.float32)
    return scores + bias


def _pass1_body(emb_ref, w_ref, b_ref, norm_ref, m_scr, s_scr):
    j = pl.program_id(0)

    @pl.when(j == 0)
    def _():
        m_scr[...] = jnp.full((1, B), -jnp.inf, jnp.float32)
        s_scr[...] = jnp.zeros((1, B), jnp.float32)

    scores = _scores_tile(emb_ref, w_ref, b_ref)

    m_old = m_scr[...]
    m_new = jnp.maximum(m_old, jnp.max(scores, axis=0, keepdims=True))
    s = (s_scr[...] * jnp.exp(m_old - m_new)
         + jnp.sum(jnp.exp(scores - m_new), axis=0, keepdims=True))
    m_scr[...] = m_new
    s_scr[...] = s

    @pl.when(j == NP - 1)
    def _():
        norm_ref[...] = m_new + jnp.log(s)


def _pass1(embeds, w, b2):
    return pl.pallas_call(
        _pass1_body,
        grid=(NP,),
        in_specs=[
            pl.BlockSpec((D, B), lambda j: (0, 0)),
            pl.BlockSpec((D, VT), lambda j: (0, j)),
            pl.BlockSpec((1, VT), lambda j: (0, j)),
        ],
        out_specs=pl.BlockSpec((1, B), lambda j: (0, 0)),
        out_shape=jax.ShapeDtypeStruct((1, B), jnp.float32),
        scratch_shapes=[
            pltpu.VMEM((1, B), jnp.float32),
            pltpu.VMEM((1, B), jnp.float32),
        ],
    )(embeds, w, b2)


def _pass2_body(emb_ref, w_ref, b_ref, norm_ref, out_ref):
    scores = _scores_tile(emb_ref, w_ref, b_ref)
    out_ref[...] = scores - norm_ref[...]


def _pass2(embeds, w, b2, norm):
    return pl.pallas_call(
        _pass2_body,
        grid=(NV,),
        in_specs=[
            pl.BlockSpec((D, B), lambda j: (0, 0)),
            pl.BlockSpec((D, VT), lambda j: (0, j)),
            pl.BlockSpec((1, VT), lambda j: (0, j)),
            pl.BlockSpec((1, B), lambda j: (0, 0)),
        ],
        out_specs=pl.BlockSpec((VT, B), lambda j: (j, 0)),
        out_shape=jax.ShapeDtypeStruct((V, B), jnp.float32),
    )(embeds, w, b2, norm)


def kernel(inputs, emb_table, W, b):
    embeds = _sc_gather(emb_table.T.reshape(-1), inputs)
    wt = jnp.pad(W.T, ((0, 0), (0, VP - V)))
    b2 = jnp.pad(b.reshape(1, V), ((0, 0), (0, VP - V)),
                 constant_values=-1e30)
    norm = _pass1(embeds, wt, b2)
    out_t = _pass2(embeds, wt, b2, norm)
    return out_t.T
